# baseline (speedup 1.0000x reference)
import functools

import jax
import jax.numpy as jnp
from jax import lax
from jax.experimental import pallas as pl
from jax.experimental.pallas import tpu as pltpu
from jax.experimental.pallas import tpu_sc as plsc

BATCH = 16384
NF = 32
NW = 32
B_PER_W = BATCH // NW
CT = 128
NCH = B_PER_W // CT
SR = 112
ER = CT - SR


def _gather_kernel(user_hbm, item_hbm, t_ug, t_ig, t_um, t_im,
                   o_ug, o_ig, o_um, o_im,
                   idx_u, idx_i, r_ug, r_ig, r_um, r_im, sem_s, sem_e):
  wid = lax.axis_index("s") * 2 + lax.axis_index("c")
  base = wid * B_PER_W
  pltpu.sync_copy(user_hbm.at[pl.ds(base, B_PER_W)], idx_u)
  pltpu.sync_copy(item_hbm.at[pl.ds(base, B_PER_W)], idx_i)

  def chunk(c, carry):
    cb = c * CT
    for g in range(CT // 16):
      uvec = idx_u[pl.ds(cb + g * 16, 16)]
      ivec = idx_i[pl.ds(cb + g * 16, 16)]
      for k in range(16):
        r = g * 16 + k
        u = uvec[k]
        v = ivec[k]
        if r < SR:
          dst = pl.ds(r, 1)
          pltpu.async_copy(t_ug.at[pl.ds(u, 1)], r_ug.at[dst], sem_s)
          pltpu.async_copy(t_ig.at[pl.ds(v, 1)], r_ig.at[dst], sem_s)
          pltpu.async_copy(t_um.at[pl.ds(u, 1)], r_um.at[dst], sem_s)
          pltpu.async_copy(t_im.at[pl.ds(v, 1)], r_im.at[dst], sem_s)
        else:
          dst = pl.ds(base + cb + r, 1)
          pltpu.async_copy(t_ug.at[pl.ds(u, 1)], o_ug.at[dst], sem_e)
          pltpu.async_copy(t_ig.at[pl.ds(v, 1)], o_ig.at[dst], sem_e)
          pltpu.async_copy(t_um.at[pl.ds(u, 1)], o_um.at[dst], sem_e)
          pltpu.async_copy(t_im.at[pl.ds(v, 1)], o_im.at[dst], sem_e)
    def drain_s(i, c2):
      src = pl.ds(0, 1)
      dst = pl.ds(i, 1)
      pltpu.make_async_copy(t_ug.at[src], r_ug.at[dst], sem_s).wait()
      pltpu.make_async_copy(t_ig.at[src], r_ig.at[dst], sem_s).wait()
      pltpu.make_async_copy(t_um.at[src], r_um.at[dst], sem_s).wait()
      pltpu.make_async_copy(t_im.at[src], r_im.at[dst], sem_s).wait()
      return c2
    lax.fori_loop(0, SR, drain_s, 0)
    out_slc = pl.ds(base + cb, SR)
    buf_slc = pl.ds(0, SR)
    pltpu.sync_copy(r_ug.at[buf_slc], o_ug.at[out_slc])
    pltpu.sync_copy(r_ig.at[buf_slc], o_ig.at[out_slc])
    pltpu.sync_copy(r_um.at[buf_slc], o_um.at[out_slc])
    pltpu.sync_copy(r_im.at[buf_slc], o_im.at[out_slc])
    return carry

  lax.fori_loop(0, NCH, chunk, 0)

  def drain_e(i, c2):
    src = pl.ds(0, 1)
    dst = pl.ds(base + i, 1)
    pltpu.make_async_copy(t_ug.at[src], o_ug.at[dst], sem_e).wait()
    pltpu.make_async_copy(t_ig.at[src], o_ig.at[dst], sem_e).wait()
    pltpu.make_async_copy(t_um.at[src], o_um.at[dst], sem_e).wait()
    pltpu.make_async_copy(t_im.at[src], o_im.at[dst], sem_e).wait()
    return c2
  lax.fori_loop(0, NCH * ER, drain_e, 0)


_row_t = jax.ShapeDtypeStruct((BATCH, NF), jnp.float32)

_gather = functools.partial(
    pl.kernel,
    out_type=(_row_t, _row_t, _row_t, _row_t),
    mesh=plsc.VectorSubcoreMesh(core_axis_name="c", subcore_axis_name="s"),
    scratch_types=[
        pltpu.VMEM((B_PER_W,), jnp.int32),
        pltpu.VMEM((B_PER_W,), jnp.int32),
        pltpu.VMEM((SR, NF), jnp.float32),
        pltpu.VMEM((SR, NF), jnp.float32),
        pltpu.VMEM((SR, NF), jnp.float32),
        pltpu.VMEM((SR, NF), jnp.float32),
        pltpu.SemaphoreType.DMA,
        pltpu.SemaphoreType.DMA,
    ],
)(_gather_kernel)


BB = 2048


def _dense_kernel(ug, ig, um, im, w0u, w0i, b0, w1, b1, w2, b2, w3, b3,
                  wog, woh, bo, out):
  h = jnp.maximum(
      jnp.dot(um[...], w0u[...], preferred_element_type=jnp.float32)
      + jnp.dot(im[...], w0i[...], preferred_element_type=jnp.float32)
      + b0[...][None, :], 0.0)
  h = jnp.maximum(
      jnp.dot(h, w1[...], preferred_element_type=jnp.float32)
      + b1[...][None, :], 0.0)
  h = jnp.maximum(
      jnp.dot(h, w2[...], preferred_element_type=jnp.float32)
      + b2[...][None, :], 0.0)
  h = jnp.maximum(
      jnp.dot(h, w3[...], preferred_element_type=jnp.float32)
      + b3[...][None, :], 0.0)
  gmf = ug[...] * ig[...]
  out[...] = (jnp.sum(gmf * wog[...][None, :], axis=1)
              + jnp.sum(h * woh[...][None, :], axis=1)
              + bo[0])


def _full2d(shape):
  return pl.BlockSpec(shape, lambda i: (0, 0))


def _full1d(shape):
  return pl.BlockSpec(shape, lambda i: (0,))


def kernel(user, item, user_emb_gmf, item_emb_gmf, user_emb_mlp, item_emb_mlp,
           W0, b0, W1, b1, W2, b2, W3, b3, Wo, bo):
  ug, ig, um, im = _gather(user.astype(jnp.int32), item.astype(jnp.int32),
                           user_emb_gmf, item_emb_gmf,
                           user_emb_mlp, item_emb_mlp)

  w0u = W0[:, :NF].T
  w0i = W0[:, NF:].T
  w1 = W1.T
  w2 = W2.T
  w3 = W3.T
  wog = Wo[0, :NF]
  woh = Wo[0, NF:]

  grid = BATCH // BB
  row_spec = pl.BlockSpec((BB, NF), lambda i: (i, 0))
  out = pl.pallas_call(
      _dense_kernel,
      grid=(grid,),
      in_specs=[
          row_spec, row_spec, row_spec, row_spec,
          _full2d(w0u.shape), _full2d(w0i.shape), _full1d(b0.shape),
          _full2d(w1.shape), _full1d(b1.shape),
          _full2d(w2.shape), _full1d(b2.shape),
          _full2d(w3.shape), _full1d(b3.shape),
          _full1d(wog.shape), _full1d(woh.shape), _full1d(bo.shape),
      ],
      out_specs=pl.BlockSpec((BB,), lambda i: (i,)),
      out_shape=jax.ShapeDtypeStruct((BATCH,), jnp.float32),
  )(ug, ig, um, im, w0u, w0i, b0, w1, b1, w2, b2, w3, b3, wog, woh, bo)
  return out


# speedup vs baseline: 1.1460x; 1.1460x over previous
import functools

import jax
import jax.numpy as jnp
from jax import lax
from jax.experimental import pallas as pl
from jax.experimental.pallas import tpu as pltpu
from jax.experimental.pallas import tpu_sc as plsc

BATCH = 16384
NF = 32
NW = 32
B_PER_W = BATCH // NW
CT = 128
NCH = B_PER_W // CT
SR = CT


def _gather_kernel(user_hbm, item_hbm, t_ug, t_ig, t_um, t_im,
                   o_ug, o_ig, o_um, o_im,
                   idx_u, idx_i, r_ug, r_ig, r_um, r_im, sem_s):
  wid = lax.axis_index("s") * 2 + lax.axis_index("c")
  base = wid * B_PER_W
  pltpu.sync_copy(user_hbm.at[pl.ds(base, B_PER_W)], idx_u)
  pltpu.sync_copy(item_hbm.at[pl.ds(base, B_PER_W)], idx_i)

  def chunk(c, carry):
    cb = c * CT
    for g in range(CT // 16):
      uvec = idx_u[pl.ds(cb + g * 16, 16)]
      ivec = idx_i[pl.ds(cb + g * 16, 16)]
      for k in range(16):
        r = g * 16 + k
        u = uvec[k]
        v = ivec[k]
        dst = pl.ds(r, 1)
        pltpu.async_copy(t_ug.at[pl.ds(u, 1)], r_ug.at[dst], sem_s)
        pltpu.async_copy(t_ig.at[pl.ds(v, 1)], r_ig.at[dst], sem_s)
        pltpu.async_copy(t_um.at[pl.ds(u, 1)], r_um.at[dst], sem_s)
        pltpu.async_copy(t_im.at[pl.ds(v, 1)], r_im.at[dst], sem_s)
    def drain_s(i, c2):
      src = pl.ds(0, 1)
      dst = pl.ds(i, 1)
      pltpu.make_async_copy(t_ug.at[src], r_ug.at[dst], sem_s).wait()
      pltpu.make_async_copy(t_ig.at[src], r_ig.at[dst], sem_s).wait()
      pltpu.make_async_copy(t_um.at[src], r_um.at[dst], sem_s).wait()
      pltpu.make_async_copy(t_im.at[src], r_im.at[dst], sem_s).wait()
      return c2
    lax.fori_loop(0, SR, drain_s, 0)
    out_slc = pl.ds(base + cb, SR)
    buf_slc = pl.ds(0, SR)
    pltpu.sync_copy(r_ug.at[buf_slc], o_ug.at[out_slc])
    pltpu.sync_copy(r_ig.at[buf_slc], o_ig.at[out_slc])
    pltpu.sync_copy(r_um.at[buf_slc], o_um.at[out_slc])
    pltpu.sync_copy(r_im.at[buf_slc], o_im.at[out_slc])
    return carry

  lax.fori_loop(0, NCH, chunk, 0)


_row_t = jax.ShapeDtypeStruct((BATCH, NF), jnp.float32)

_gather = functools.partial(
    pl.kernel,
    out_type=(_row_t, _row_t, _row_t, _row_t),
    mesh=plsc.VectorSubcoreMesh(core_axis_name="c", subcore_axis_name="s"),
    scratch_types=[
        pltpu.VMEM((B_PER_W,), jnp.int32),
        pltpu.VMEM((B_PER_W,), jnp.int32),
        pltpu.VMEM((SR, NF), jnp.float32),
        pltpu.VMEM((SR, NF), jnp.float32),
        pltpu.VMEM((SR, NF), jnp.float32),
        pltpu.VMEM((SR, NF), jnp.float32),
        pltpu.SemaphoreType.DMA,
    ],
)(_gather_kernel)


BB = 2048


def _dense_kernel(ug, ig, um, im, w0u, w0i, b0, w1, b1, w2, b2, w3, b3,
                  wog, woh, bo, out):
  h = jnp.maximum(
      jnp.dot(um[...], w0u[...], preferred_element_type=jnp.float32)
      + jnp.dot(im[...], w0i[...], preferred_element_type=jnp.float32)
      + b0[...][None, :], 0.0)
  h = jnp.maximum(
      jnp.dot(h, w1[...], preferred_element_type=jnp.float32)
      + b1[...][None, :], 0.0)
  h = jnp.maximum(
      jnp.dot(h, w2[...], preferred_element_type=jnp.float32)
      + b2[...][None, :], 0.0)
  h = jnp.maximum(
      jnp.dot(h, w3[...], preferred_element_type=jnp.float32)
      + b3[...][None, :], 0.0)
  gmf = ug[...] * ig[...]
  out[...] = (jnp.sum(gmf * wog[...][None, :], axis=1)
              + jnp.sum(h * woh[...][None, :], axis=1)
              + bo[0])


def _full2d(shape):
  return pl.BlockSpec(shape, lambda i: (0, 0))


def _full1d(shape):
  return pl.BlockSpec(shape, lambda i: (0,))


def kernel(user, item, user_emb_gmf, item_emb_gmf, user_emb_mlp, item_emb_mlp,
           W0, b0, W1, b1, W2, b2, W3, b3, Wo, bo):
  ug, ig, um, im = _gather(user.astype(jnp.int32), item.astype(jnp.int32),
                           user_emb_gmf, item_emb_gmf,
                           user_emb_mlp, item_emb_mlp)

  w0u = W0[:, :NF].T
  w0i = W0[:, NF:].T
  w1 = W1.T
  w2 = W2.T
  w3 = W3.T
  wog = Wo[0, :NF]
  woh = Wo[0, NF:]

  grid = BATCH // BB
  row_spec = pl.BlockSpec((BB, NF), lambda i: (i, 0))
  out = pl.pallas_call(
      _dense_kernel,
      grid=(grid,),
      in_specs=[
          row_spec, row_spec, row_spec, row_spec,
          _full2d(w0u.shape), _full2d(w0i.shape), _full1d(b0.shape),
          _full2d(w1.shape), _full1d(b1.shape),
          _full2d(w2.shape), _full1d(b2.shape),
          _full2d(w3.shape), _full1d(b3.shape),
          _full1d(wog.shape), _full1d(woh.shape), _full1d(bo.shape),
      ],
      out_specs=pl.BlockSpec((BB,), lambda i: (i,)),
      out_shape=jax.ShapeDtypeStruct((BATCH,), jnp.float32),
  )(ug, ig, um, im, w0u, w0i, b0, w1, b1, w2, b2, w3, b3, wog, woh, bo)
  return out
